# baseline (speedup 1.0000x reference)
import functools

import jax
import jax.numpy as jnp
from jax import lax
from jax.experimental import pallas as pl
from jax.experimental.pallas import tpu as pltpu
from jax.experimental.pallas import tpu_sc as plsc

N_ROWS = 1024 * 768
IN_ROW = 16
OUT_ROW = 64
CHUNK = 1024


@functools.cache
def _sc_upsample():
    info = plsc.get_sparse_core_info()
    num_cores, num_subcores = info.num_cores, info.num_subcores
    num_workers = num_cores * num_subcores
    rows_per_w = N_ROWS // num_workers
    n_chunks = rows_per_w // CHUNK
    mesh = plsc.VectorSubcoreMesh(core_axis_name="c", subcore_axis_name="s")

    @functools.partial(
        pl.kernel,
        mesh=mesh,
        out_type=jax.ShapeDtypeStruct((N_ROWS * OUT_ROW,), jnp.float32),
        scratch_types=[
            pltpu.VMEM((CHUNK * IN_ROW,), jnp.float32),
            pltpu.VMEM((CHUNK * OUT_ROW,), jnp.float32),
        ],
    )
    def upsample(in_hbm, out_hbm, in_v, out_v):
        wid = lax.axis_index("s") * num_cores + lax.axis_index("c")
        row0 = wid * rows_per_w
        lane = lax.iota(jnp.int32, 16)
        pats = [((lane >> 1) & 3) + 4 * k for k in range(4)]

        def chunk_body(ci, carry):
            base = row0 + ci * CHUNK
            pltpu.sync_copy(in_hbm.at[pl.ds(base * IN_ROW, CHUNK * IN_ROW)], in_v)

            def row_body(r, c2):
                v_in = in_v[pl.ds(r * IN_ROW, 16)]
                for k in range(4):
                    v = jnp.take_along_axis(v_in, pats[k], axis=0)
                    out_v[pl.ds(r * OUT_ROW + k * 16, 16)] = v
                return c2

            lax.fori_loop(0, CHUNK, row_body, 0)
            pltpu.sync_copy(out_v, out_hbm.at[pl.ds(base * OUT_ROW, CHUNK * OUT_ROW)])
            return carry

        lax.fori_loop(0, n_chunks, chunk_body, 0)

    return upsample


def kernel(arg0_1):
    flat = arg0_1.reshape(-1)
    out = _sc_upsample()(flat)
    return (out.reshape(1024, 768, 8, 8),)


# speedup vs baseline: 1.0122x; 1.0122x over previous
import functools

import jax
import jax.numpy as jnp
from jax import lax
from jax.experimental import pallas as pl
from jax.experimental.pallas import tpu as pltpu
from jax.experimental.pallas import tpu_sc as plsc

N_ROWS = 1024 * 768
IN_ROW = 16
OUT_ROW = 64
CHUNK = 1024


@functools.cache
def _sc_upsample():
    info = plsc.get_sparse_core_info()
    num_cores, num_subcores = info.num_cores, info.num_subcores
    num_workers = num_cores * num_subcores
    rows_per_w = N_ROWS // num_workers
    n_chunks = rows_per_w // CHUNK
    mesh = plsc.VectorSubcoreMesh(core_axis_name="c", subcore_axis_name="s")

    @functools.partial(
        pl.kernel,
        mesh=mesh,
        out_type=jax.ShapeDtypeStruct((N_ROWS * OUT_ROW,), jnp.float32),
        scratch_types=[
            pltpu.VMEM((CHUNK * IN_ROW,), jnp.float32),
            pltpu.VMEM((CHUNK * OUT_ROW,), jnp.float32),
        ],
    )
    def upsample(in_hbm, out_hbm, in_v, out_v):
        wid = lax.axis_index("s") * num_cores + lax.axis_index("c")
        row0 = wid * rows_per_w
        lane = lax.iota(jnp.int32, 16)
        pats = [((lane >> 1) & 3) + 4 * k for k in range(4)]

        def chunk_body(ci, carry):
            base = row0 + ci * CHUNK
            pltpu.sync_copy(in_hbm.at[pl.ds(base * IN_ROW, CHUNK * IN_ROW)], in_v)

            @plsc.parallel_loop(0, CHUNK, unroll=8)
            def _row_body(r):
                v_in = in_v[pl.ds(r * IN_ROW, 16)]
                for k in range(4):
                    v = jnp.take_along_axis(v_in, pats[k], axis=0)
                    out_v[pl.ds(r * OUT_ROW + k * 16, 16)] = v
            pltpu.sync_copy(out_v, out_hbm.at[pl.ds(base * OUT_ROW, CHUNK * OUT_ROW)])
            return carry

        lax.fori_loop(0, n_chunks, chunk_body, 0)

    return upsample


def kernel(arg0_1):
    flat = arg0_1.reshape(-1)
    out = _sc_upsample()(flat)
    return (out.reshape(1024, 768, 8, 8),)


# speedup vs baseline: 62.5621x; 61.8074x over previous
import functools

import jax
import jax.numpy as jnp
from jax import lax
from jax.experimental import pallas as pl
from jax.experimental.pallas import tpu as pltpu
from jax.experimental.pallas import tpu_sc as plsc

B = 1024
NBUF = 2


@functools.cache
def _sc_upsample():
    info = plsc.get_sparse_core_info()
    num_cores, num_subcores = info.num_cores, info.num_subcores
    num_workers = num_cores * num_subcores
    b_per_w = B // num_workers
    mesh = plsc.VectorSubcoreMesh(core_axis_name="c", subcore_axis_name="s")

    @functools.partial(
        pl.kernel,
        mesh=mesh,
        out_type=jax.ShapeDtypeStruct((B, 8, 6, 8, 128), jnp.float32),
        scratch_types=[
            [pltpu.VMEM((4, 6, 4, 128), jnp.float32) for _ in range(NBUF)],
            [pltpu.SemaphoreType.DMA for _ in range(NBUF)],
            [pltpu.SemaphoreType.DMA for _ in range(NBUF)],
        ],
    )
    def upsample(in_hbm, out_hbm, in_bufs, in_sems, out_sems):
        wid = lax.axis_index("s") * num_cores + lax.axis_index("c")
        b0 = wid * b_per_w

        def in_copy(b, buf):
            return pltpu.make_async_copy(in_hbm.at[b], in_bufs[buf], in_sems[buf])

        def out_copies(b, buf):
            return [
                pltpu.make_async_copy(
                    in_bufs[buf].at[ip // 2, :, jp // 2, :],
                    out_hbm.at[b, ip, :, jp, :],
                    out_sems[buf],
                )
                for ip in range(8)
                for jp in range(8)
            ]

        for buf in range(NBUF):
            in_copy(b0 + buf, buf).start()

        def body(k, carry):
            b = b0 + k * NBUF
            for buf in range(NBUF):
                in_copy(b + buf, buf).wait()

                @pl.when(k > 0)
                def _():
                    for cp in out_copies(b + buf - NBUF, buf):
                        cp.wait()

                for cp in out_copies(b + buf, buf):
                    cp.start()

                @pl.when(k < b_per_w // NBUF - 1)
                def _():
                    in_copy(b + buf + NBUF, buf).start()
            return carry

        lax.fori_loop(0, b_per_w // NBUF, body, 0)
        for buf in range(NBUF):
            for cp in out_copies(b0 + b_per_w - NBUF + buf, buf):
                cp.wait()

    return upsample


def kernel(arg0_1):
    x5 = arg0_1.reshape(B, 6, 128, 4, 4).transpose(0, 3, 1, 4, 2)
    o5 = _sc_upsample()(x5)
    out = o5.transpose(0, 2, 4, 1, 3).reshape(B, 768, 8, 8)
    return (out,)


# speedup vs baseline: 63.6652x; 1.0176x over previous
import functools

import jax
import jax.numpy as jnp
from jax import lax
from jax.experimental import pallas as pl
from jax.experimental.pallas import tpu as pltpu
from jax.experimental.pallas import tpu_sc as plsc

B = 1024
NBUF = 2


@functools.cache
def _sc_upsample():
    info = plsc.get_sparse_core_info()
    num_cores, num_subcores = info.num_cores, info.num_subcores
    num_workers = num_cores * num_subcores
    b_per_w = B // num_workers
    mesh = plsc.VectorSubcoreMesh(core_axis_name="c", subcore_axis_name="s")

    @functools.partial(
        pl.kernel,
        mesh=mesh,
        out_type=jax.ShapeDtypeStruct((B, 8, 6, 8, 128), jnp.float32),
        scratch_types=[
            [pltpu.VMEM((4, 6, 4, 128), jnp.float32) for _ in range(NBUF)],
            [pltpu.VMEM((4, 6, 8, 128), jnp.float32) for _ in range(NBUF)],
            [pltpu.SemaphoreType.DMA for _ in range(NBUF)],
            [pltpu.SemaphoreType.DMA for _ in range(NBUF)],
        ],
    )
    def upsample(in_hbm, out_hbm, in_bufs, st_bufs, in_sems, out_sems):
        wid = lax.axis_index("s") * num_cores + lax.axis_index("c")
        b0 = wid * b_per_w

        def in_copy(b, buf):
            return pltpu.make_async_copy(in_hbm.at[b], in_bufs[buf], in_sems[buf])

        def out_copies(b, buf):
            return [
                pltpu.make_async_copy(
                    st_bufs[buf].at[ip // 2],
                    out_hbm.at[b, ip],
                    out_sems[buf],
                )
                for ip in range(8)
            ]

        for buf in range(NBUF):
            in_copy(b0 + buf, buf).start()

        def body(k, carry):
            b = b0 + k * NBUF
            for buf in range(NBUF):
                in_copy(b + buf, buf).wait()

                @pl.when(k > 0)
                def _():
                    for cp in out_copies(b + buf - NBUF, buf):
                        cp.wait()

                @plsc.parallel_loop(0, 24, unroll=4)
                def _rows(r):
                    i = r // 6
                    ct = r % 6
                    for j in range(4):
                        for l in range(8):
                            v = in_bufs[buf][i, ct, j, pl.ds(l * 16, 16)]
                            st_bufs[buf][i, ct, 2 * j, pl.ds(l * 16, 16)] = v
                            st_bufs[buf][i, ct, 2 * j + 1, pl.ds(l * 16, 16)] = v

                for cp in out_copies(b + buf, buf):
                    cp.start()

                @pl.when(k < b_per_w // NBUF - 1)
                def _():
                    in_copy(b + buf + NBUF, buf).start()
            return carry

        lax.fori_loop(0, b_per_w // NBUF, body, 0)
        for buf in range(NBUF):
            for cp in out_copies(b0 + b_per_w - NBUF + buf, buf):
                cp.wait()

    return upsample


def kernel(arg0_1):
    x5 = arg0_1.reshape(B, 6, 128, 4, 4).transpose(0, 3, 1, 4, 2)
    o5 = _sc_upsample()(x5)
    out = o5.transpose(0, 2, 4, 1, 3).reshape(B, 768, 8, 8)
    return (out,)
